# Initial kernel scaffold; baseline (speedup 1.0000x reference)
import jax
import jax.numpy as jnp
from jax.experimental import pallas as pl


def kernel(edge_attr, batch_e, W1, b1, W2, b2):
    raise NotImplementedError("write your pallas kernel here")


# speedup vs baseline: 22.0027x; 22.0027x over previous
import functools

import jax
import jax.numpy as jnp
from jax.experimental import pallas as pl
from jax.experimental.pallas import tpu as pltpu

_B = 512
_F = 16


def _pool_body(seg_ref, x_ref, w1_ref, b1_ref, w2r_ref, b2_ref, out_ref,
               acc_ref):
    i = pl.program_id(0)

    @pl.when(i == 0)
    def _init():
        acc_ref[...] = jnp.zeros_like(acc_ref)

    x = x_ref[...]
    h = jnp.tanh(
        jnp.dot(x, w1_ref[...], preferred_element_type=jnp.float32)
        + b1_ref[...])
    s = jnp.sum(h * w2r_ref[...], axis=1, keepdims=True) + b2_ref[...]
    ex = jnp.exp(s)
    y = jnp.concatenate([x * ex, ex], axis=1)

    seg = seg_ref[0, 0, :]
    ids = jax.lax.broadcasted_iota(jnp.int32, (_B, 1), 0)
    oh = (ids == seg[None, :]).astype(jnp.float32)
    acc_ref[...] += jnp.dot(oh, y, preferred_element_type=jnp.float32)

    @pl.when(i == pl.num_programs(0) - 1)
    def _fin():
        acc = acc_ref[...]
        den = acc[:, _F:_F + 1]
        den = jnp.where(den == 0.0, 1.0, den)
        out_ref[...] = acc[:, :_F] / den


def kernel(edge_attr, batch_e, W1, b1, W2, b2):
    E, F = edge_attr.shape
    H = W1.shape[1]
    K = 4000 if E % 4000 == 0 else 8
    nblk = E // K

    seg3 = batch_e.astype(jnp.int32).reshape(nblk, 1, K)
    b1r = b1.reshape(1, H).astype(jnp.float32)
    w2r = W2.reshape(1, H).astype(jnp.float32)
    b2r = b2.reshape(1, 1).astype(jnp.float32)

    grid_spec = pltpu.PrefetchScalarGridSpec(
        num_scalar_prefetch=0,
        grid=(nblk,),
        in_specs=[
            pl.BlockSpec((1, 1, K), lambda i: (i, 0, 0)),
            pl.BlockSpec((K, F), lambda i: (i, 0)),
            pl.BlockSpec((F, H), lambda i: (0, 0)),
            pl.BlockSpec((1, H), lambda i: (0, 0)),
            pl.BlockSpec((1, H), lambda i: (0, 0)),
            pl.BlockSpec((1, 1), lambda i: (0, 0)),
        ],
        out_specs=pl.BlockSpec((_B, _F), lambda i: (0, 0)),
        scratch_shapes=[pltpu.VMEM((_B, _F + 1), jnp.float32)],
    )
    return pl.pallas_call(
        _pool_body,
        grid_spec=grid_spec,
        out_shape=jax.ShapeDtypeStruct((_B, _F), jnp.float32),
        compiler_params=pltpu.CompilerParams(
            dimension_semantics=("arbitrary",)),
    )(seg3, edge_attr.astype(jnp.float32), W1.astype(jnp.float32),
      b1r, w2r, b2r)


# speedup vs baseline: 31.6937x; 1.4404x over previous
import functools

import jax
import jax.numpy as jnp
from jax.experimental import pallas as pl
from jax.experimental.pallas import tpu as pltpu

_B = 512
_F = 16
_W = 40


def _pool_body(starts_ref, ends_ref, seg_ref, x_ref, w1_ref, b1_ref, w2r_ref,
               b2_ref, out_ref, acc_ref):
    i = pl.program_id(0)

    @pl.when(i == 0)
    def _init():
        acc_ref[...] = jnp.zeros_like(acc_ref)

    x = x_ref[...]
    h = jnp.tanh(
        jnp.dot(x, w1_ref[...], preferred_element_type=jnp.float32)
        + b1_ref[...])
    s = jnp.sum(h * w2r_ref[...], axis=1, keepdims=True) + b2_ref[...]
    ex = jnp.exp(s)
    y = jnp.concatenate([x * ex, ex], axis=1)

    seg = seg_ref[0, 0, :]
    lo = starts_ref[i]
    lo_al = (lo // 8) * 8
    hi = ends_ref[i]

    @pl.when(hi - lo_al < _W)
    def _fast():
        ids = jax.lax.broadcasted_iota(jnp.int32, (_W, 1), 0) + lo_al
        oh = (ids == seg[None, :]).astype(jnp.float32)
        part = jnp.dot(oh, y, preferred_element_type=jnp.float32)
        cur = acc_ref[pl.ds(lo_al, _W), :]
        acc_ref[pl.ds(lo_al, _W), :] = cur + part

    @pl.when(hi - lo_al >= _W)
    def _slow():
        ids = jax.lax.broadcasted_iota(jnp.int32, (_B, 1), 0)
        oh = (ids == seg[None, :]).astype(jnp.float32)
        part = jnp.dot(oh, y, preferred_element_type=jnp.float32)
        acc_ref[:_B, :] = acc_ref[:_B, :] + part

    @pl.when(i == pl.num_programs(0) - 1)
    def _fin():
        acc = acc_ref[:_B, :]
        den = acc[:, _F:_F + 1]
        den = jnp.where(den == 0.0, 1.0, den)
        out_ref[...] = acc[:, :_F] / den


def kernel(edge_attr, batch_e, W1, b1, W2, b2):
    E, F = edge_attr.shape
    H = W1.shape[1]
    K = 4000 if E % 4000 == 0 else 8
    nblk = E // K

    seg = batch_e.astype(jnp.int32)
    seg3 = seg.reshape(nblk, 1, K)
    starts = seg[::K]
    ends = seg[K - 1::K]
    b1r = b1.reshape(1, H).astype(jnp.float32)
    w2r = W2.reshape(1, H).astype(jnp.float32)
    b2r = b2.reshape(1, 1).astype(jnp.float32)

    grid_spec = pltpu.PrefetchScalarGridSpec(
        num_scalar_prefetch=2,
        grid=(nblk,),
        in_specs=[
            pl.BlockSpec((1, 1, K), lambda i, *_: (i, 0, 0)),
            pl.BlockSpec((K, F), lambda i, *_: (i, 0)),
            pl.BlockSpec((F, H), lambda i, *_: (0, 0)),
            pl.BlockSpec((1, H), lambda i, *_: (0, 0)),
            pl.BlockSpec((1, H), lambda i, *_: (0, 0)),
            pl.BlockSpec((1, 1), lambda i, *_: (0, 0)),
        ],
        out_specs=pl.BlockSpec((_B, _F), lambda i, *_: (0, 0)),
        scratch_shapes=[pltpu.VMEM((_B + _W, _F + 1), jnp.float32)],
    )
    return pl.pallas_call(
        _pool_body,
        grid_spec=grid_spec,
        out_shape=jax.ShapeDtypeStruct((_B, _F), jnp.float32),
        compiler_params=pltpu.CompilerParams(
            dimension_semantics=("arbitrary",)),
    )(starts, ends, seg3, edge_attr.astype(jnp.float32),
      W1.astype(jnp.float32), b1r, w2r, b2r)


# speedup vs baseline: 34.4065x; 1.0856x over previous
import functools

import jax
import jax.numpy as jnp
from jax.experimental import pallas as pl
from jax.experimental.pallas import tpu as pltpu

_B = 512
_F = 16
_W = 16


def _pool_body(starts_ref, ends_ref, seg_ref, x_ref, w1_ref, b1_ref, w2r_ref,
               b2_ref, out_ref, acc_ref):
    i = pl.program_id(0)

    @pl.when(i == 0)
    def _init():
        acc_ref[...] = jnp.zeros_like(acc_ref)

    x = x_ref[...]
    h = jnp.tanh(
        jnp.dot(x, w1_ref[...], preferred_element_type=jnp.float32)
        + b1_ref[...])
    s = jnp.sum(h * w2r_ref[...], axis=1, keepdims=True) + b2_ref[...]
    ex = jnp.exp(s)
    y = jnp.concatenate([x * ex, ex], axis=1)

    seg = seg_ref[0, 0, :]
    lo = starts_ref[i]
    lo_al = (lo // 8) * 8
    hi = ends_ref[i]
    nwin = (hi - lo_al) // _W + 1

    def _win(j, carry):
        base = lo_al + j * _W
        ids = jax.lax.broadcasted_iota(jnp.int32, (_W, 1), 0) + base
        oh = (ids == seg[None, :]).astype(jnp.float32)
        part = jnp.dot(oh, y, preferred_element_type=jnp.float32)
        cur = acc_ref[pl.ds(base, _W), :]
        acc_ref[pl.ds(base, _W), :] = cur + part
        return carry

    jax.lax.fori_loop(0, nwin, _win, 0)

    @pl.when(i == pl.num_programs(0) - 1)
    def _fin():
        acc = acc_ref[:_B, :]
        den = acc[:, _F:_F + 1]
        den = jnp.where(den == 0.0, 1.0, den)
        out_ref[...] = acc[:, :_F] / den


def kernel(edge_attr, batch_e, W1, b1, W2, b2):
    E, F = edge_attr.shape
    H = W1.shape[1]
    K = 8000 if E % 8000 == 0 else 8
    nblk = E // K

    seg = batch_e.astype(jnp.int32)
    seg3 = seg.reshape(nblk, 1, K)
    starts = seg[::K]
    ends = seg[K - 1::K]
    b1r = b1.reshape(1, H).astype(jnp.float32)
    w2r = W2.reshape(1, H).astype(jnp.float32)
    b2r = b2.reshape(1, 1).astype(jnp.float32)

    grid_spec = pltpu.PrefetchScalarGridSpec(
        num_scalar_prefetch=2,
        grid=(nblk,),
        in_specs=[
            pl.BlockSpec((1, 1, K), lambda i, *_: (i, 0, 0)),
            pl.BlockSpec((K, F), lambda i, *_: (i, 0)),
            pl.BlockSpec((F, H), lambda i, *_: (0, 0)),
            pl.BlockSpec((1, H), lambda i, *_: (0, 0)),
            pl.BlockSpec((1, H), lambda i, *_: (0, 0)),
            pl.BlockSpec((1, 1), lambda i, *_: (0, 0)),
        ],
        out_specs=pl.BlockSpec((_B, _F), lambda i, *_: (0, 0)),
        scratch_shapes=[pltpu.VMEM((_B + _W, _F + 1), jnp.float32)],
    )
    return pl.pallas_call(
        _pool_body,
        grid_spec=grid_spec,
        out_shape=jax.ShapeDtypeStruct((_B, _F), jnp.float32),
        compiler_params=pltpu.CompilerParams(
            dimension_semantics=("arbitrary",)),
    )(starts, ends, seg3, edge_attr.astype(jnp.float32),
      W1.astype(jnp.float32), b1r, w2r, b2r)


# speedup vs baseline: 44.3598x; 1.2893x over previous
import functools

import jax
import jax.numpy as jnp
from jax.experimental import pallas as pl
from jax.experimental.pallas import tpu as pltpu

_B = 512
_F = 16
_W = 16


def _pool_body(starts_ref, ends_ref, seg_ref, x_ref, w1_ref, b1_ref,
               w2rep_ref, b2_ref, out_ref, acc_ref):
    i = pl.program_id(0)

    @pl.when(i == 0)
    def _init():
        acc_ref[...] = jnp.zeros_like(acc_ref)

    x = x_ref[...]
    h = jnp.tanh(
        jnp.dot(x, w1_ref[...], preferred_element_type=jnp.float32)
        + b1_ref[...])
    s = (jnp.dot(h, w2rep_ref[...], preferred_element_type=jnp.float32)
         + b2_ref[...])
    ex = jnp.exp(s)
    y = x * ex

    seg = seg_ref[0, 0, :]
    lo = starts_ref[i]
    lo_al = (lo // 8) * 8
    hi = ends_ref[i]
    nwin = (hi - lo_al) // _W + 1

    def _win(j, carry):
        base = lo_al + j * _W
        ids = jax.lax.broadcasted_iota(jnp.int32, (_W, 1), 0) + base
        oh = (ids == seg[None, :]).astype(jnp.float32)
        num = jnp.dot(oh, y, preferred_element_type=jnp.float32)
        den = jnp.dot(oh, ex, preferred_element_type=jnp.float32)
        cur_n = acc_ref[pl.ds(base, _W), :_F]
        acc_ref[pl.ds(base, _W), :_F] = cur_n + num
        cur_d = acc_ref[pl.ds(base, _W), _F:]
        acc_ref[pl.ds(base, _W), _F:] = cur_d + den
        return carry

    jax.lax.fori_loop(0, nwin, _win, 0)

    @pl.when(i == pl.num_programs(0) - 1)
    def _fin():
        acc = acc_ref[:_B, :]
        den = acc[:, _F:_F + 1]
        den = jnp.where(den == 0.0, 1.0, den)
        out_ref[...] = acc[:, :_F] / den


def kernel(edge_attr, batch_e, W1, b1, W2, b2):
    E, F = edge_attr.shape
    H = W1.shape[1]
    K = 8000 if E % 8000 == 0 else 8
    nblk = E // K

    seg = batch_e.astype(jnp.int32)
    seg3 = seg.reshape(nblk, 1, K)
    starts = seg[::K]
    ends = seg[K - 1::K]
    b1r = b1.reshape(1, H).astype(jnp.float32)
    w2rep = jnp.tile(W2.astype(jnp.float32), (1, _F))
    b2r = b2.reshape(1, 1).astype(jnp.float32)

    grid_spec = pltpu.PrefetchScalarGridSpec(
        num_scalar_prefetch=2,
        grid=(nblk,),
        in_specs=[
            pl.BlockSpec((1, 1, K), lambda i, *_: (i, 0, 0)),
            pl.BlockSpec((K, F), lambda i, *_: (i, 0)),
            pl.BlockSpec((F, H), lambda i, *_: (0, 0)),
            pl.BlockSpec((1, H), lambda i, *_: (0, 0)),
            pl.BlockSpec((H, _F), lambda i, *_: (0, 0)),
            pl.BlockSpec((1, 1), lambda i, *_: (0, 0)),
        ],
        out_specs=pl.BlockSpec((_B, _F), lambda i, *_: (0, 0)),
        scratch_shapes=[pltpu.VMEM((_B + _W, 2 * _F), jnp.float32)],
    )
    return pl.pallas_call(
        _pool_body,
        grid_spec=grid_spec,
        out_shape=jax.ShapeDtypeStruct((_B, _F), jnp.float32),
        compiler_params=pltpu.CompilerParams(
            dimension_semantics=("arbitrary",)),
    )(starts, ends, seg3, edge_attr.astype(jnp.float32),
      W1.astype(jnp.float32), b1r, w2rep, b2r)


# speedup vs baseline: 45.1268x; 1.0173x over previous
import functools

import jax
import jax.numpy as jnp
from jax.experimental import pallas as pl
from jax.experimental.pallas import tpu as pltpu

_B = 512
_F = 16
_W = 16


def _pool_body(starts_ref, ends_ref, seg_ref, x_ref, w1_ref, b1_ref,
               w2rep_ref, b2_ref, out_ref, acc_ref):
    i = pl.program_id(0)

    @pl.when(i == 0)
    def _init():
        acc_ref[...] = jnp.zeros_like(acc_ref)

    x = x_ref[...]
    h = jnp.tanh(
        jnp.dot(x, w1_ref[...], preferred_element_type=jnp.float32)
        + b1_ref[...])
    s = (jnp.dot(h, w2rep_ref[...], preferred_element_type=jnp.float32)
         + b2_ref[...])
    ex = jnp.exp(s)
    y = x * ex

    seg = seg_ref[0, 0, :]
    lo = starts_ref[i]
    lo_al = (lo // 8) * 8
    hi = ends_ref[i]
    nwin = (hi - lo_al) // _W + 1

    def _win(j):
        base = lo_al + j * _W
        ids = jax.lax.broadcasted_iota(jnp.int32, (_W, 1), 0) + base
        oh = (ids == seg[None, :]).astype(jnp.float32)
        num = jnp.dot(oh, y, preferred_element_type=jnp.float32)
        den = jnp.dot(oh, ex, preferred_element_type=jnp.float32)
        cur_n = acc_ref[pl.ds(base, _W), :_F]
        acc_ref[pl.ds(base, _W), :_F] = cur_n + num
        cur_d = acc_ref[pl.ds(base, _W), _F:]
        acc_ref[pl.ds(base, _W), _F:] = cur_d + den

    _win(0)

    @pl.when(nwin > 1)
    def _rest():
        jax.lax.fori_loop(1, nwin, lambda j, c: (_win(j), c)[1], 0)

    @pl.when(i == pl.num_programs(0) - 1)
    def _fin():
        acc = acc_ref[:_B, :]
        den = acc[:, _F:_F + 1]
        den = jnp.where(den == 0.0, 1.0, den)
        out_ref[...] = acc[:, :_F] / den


def kernel(edge_attr, batch_e, W1, b1, W2, b2):
    E, F = edge_attr.shape
    H = W1.shape[1]
    K = 8000 if E % 8000 == 0 else 8
    nblk = E // K

    seg = batch_e.astype(jnp.int32)
    seg3 = seg.reshape(nblk, 1, K)
    starts = seg[::K]
    ends = seg[K - 1::K]
    b1r = b1.reshape(1, H).astype(jnp.float32)
    w2rep = jnp.tile(W2.astype(jnp.float32), (1, _F))
    b2r = b2.reshape(1, 1).astype(jnp.float32)

    grid_spec = pltpu.PrefetchScalarGridSpec(
        num_scalar_prefetch=2,
        grid=(nblk,),
        in_specs=[
            pl.BlockSpec((1, 1, K), lambda i, *_: (i, 0, 0)),
            pl.BlockSpec((K, F), lambda i, *_: (i, 0)),
            pl.BlockSpec((F, H), lambda i, *_: (0, 0)),
            pl.BlockSpec((1, H), lambda i, *_: (0, 0)),
            pl.BlockSpec((H, _F), lambda i, *_: (0, 0)),
            pl.BlockSpec((1, 1), lambda i, *_: (0, 0)),
        ],
        out_specs=pl.BlockSpec((_B, _F), lambda i, *_: (0, 0)),
        scratch_shapes=[pltpu.VMEM((_B + _W, 2 * _F), jnp.float32)],
    )
    return pl.pallas_call(
        _pool_body,
        grid_spec=grid_spec,
        out_shape=jax.ShapeDtypeStruct((_B, _F), jnp.float32),
        compiler_params=pltpu.CompilerParams(
            dimension_semantics=("arbitrary",)),
    )(starts, ends, seg3, edge_attr.astype(jnp.float32),
      W1.astype(jnp.float32), b1r, w2rep, b2r)
